# baseline (speedup 1.0000x reference)
import functools

import jax
import jax.numpy as jnp
from jax import lax
from jax.experimental import pallas as pl
from jax.experimental.pallas import tpu as pltpu
from jax.experimental.pallas import tpu_sc as plsc

B = 8192
D_IN = 1024
FEAT = 512
E = 16
CLS = 64
BT = 1024
SUB = 256

NWORK = 32
CHUNK = B // NWORK


def _moe_block(x_ref, wbb_ref, bbb_ref, wc_ref, bc_ref, we_ref, be_ref,
               coarse_ref, eid_ref, sel_ref, wcat_ref):
    @pl.when(pl.program_id(0) == 0)
    def _init():
        for e in range(E):
            wcat_ref[:, e * CLS:(e + 1) * CLS] = we_ref[e]

    wbb = wbb_ref[...]
    wc = wc_ref[...]
    wcat = wcat_ref[...]
    for s in range(BT // SUB):
        r0 = s * SUB
        xb = x_ref[r0:r0 + SUB].reshape(SUB, D_IN)
        sf = jnp.maximum(jnp.dot(xb, wbb,
                                 preferred_element_type=jnp.float32)
                         + bbb_ref[...], 0.0)
        coarse = jnp.dot(sf, wc,
                         preferred_element_type=jnp.float32) + bc_ref[...]
        coarse_ref[r0:r0 + SUB] = coarse
        eid = jnp.argmax(coarse, axis=1).astype(jnp.int32)
        eid_ref[r0:r0 + SUB] = eid[:, None]

        all_l = jnp.dot(sf, wcat,
                        preferred_element_type=jnp.float32) + be_ref[...]

        pid = eid >> 1
        sel2 = all_l[:, 0:2 * CLS]
        for p in range(1, E // 2):
            sel2 = jnp.where((pid == p)[:, None],
                             all_l[:, p * 2 * CLS:(p + 1) * 2 * CLS], sel2)
        sel_ref[r0:r0 + SUB] = jnp.where((eid & 1 == 1)[:, None],
                                         sel2[:, CLS:2 * CLS],
                                         sel2[:, 0:CLS])


def _sc_epilogue(sel1, eid1):
    mesh = plsc.VectorSubcoreMesh(core_axis_name="c", subcore_axis_name="s")

    @functools.partial(
        pl.kernel, mesh=mesh,
        out_type=[jax.ShapeDtypeStruct((B * CLS,), jnp.float32),
                  jax.ShapeDtypeStruct((B,), jnp.float32)],
        scratch_types=[pltpu.VMEM((CHUNK * CLS,), jnp.float32),
                       pltpu.VMEM((CHUNK,), jnp.int32),
                       pltpu.VMEM((CHUNK * CLS,), jnp.float32),
                       pltpu.VMEM((CHUNK,), jnp.float32)],
    )
    def k(sel_hbm, eid_hbm, lp_hbm, gp_hbm, sel_v, eid_v, lp_v, gp_v):
        wid = lax.axis_index("s") * 2 + lax.axis_index("c")
        base = wid * CHUNK
        pltpu.sync_copy(sel_hbm.at[pl.ds(base * CLS, CHUNK * CLS)], sel_v)
        pltpu.sync_copy(eid_hbm.at[pl.ds(base, CHUNK)], eid_v)
        io = lax.broadcasted_iota(jnp.int32, (16,), 0)
        big = jnp.int32(1 << 20)

        def lanered(v, op):
            for sh in (8, 4, 2, 1):
                perm = io ^ sh
                shuf = lax.gather(
                    v, perm[:, None],
                    lax.GatherDimensionNumbers(offset_dims=(),
                                               collapsed_slice_dims=(0,),
                                               start_index_map=(0,)),
                    slice_sizes=(1,),
                    mode=lax.GatherScatterMode.PROMISE_IN_BOUNDS)
                v = op(v, shuf)
            return v

        def group(g, carry):
            lc_vec = jnp.zeros((16,), jnp.int32)
            for j in range(16):
                o = (g * 16 + j) * CLS
                v0 = sel_v[pl.ds(o, 16)]
                v1 = sel_v[pl.ds(o + 16, 16)]
                v2 = sel_v[pl.ds(o + 32, 16)]
                v3 = sel_v[pl.ds(o + 48, 16)]
                m = lanered(jnp.maximum(jnp.maximum(v0, v1),
                                        jnp.maximum(v2, v3)), jnp.maximum)
                i0 = jnp.where(v0 == m, io, big)
                i1 = jnp.where(v1 == m, io + 16, big)
                i2 = jnp.where(v2 == m, io + 32, big)
                i3 = jnp.where(v3 == m, io + 48, big)
                local = lanered(jnp.minimum(jnp.minimum(i0, i1),
                                            jnp.minimum(i2, i3)), jnp.minimum)
                e0 = jnp.exp(v0 - m)
                e1 = jnp.exp(v1 - m)
                e2 = jnp.exp(v2 - m)
                e3 = jnp.exp(v3 - m)
                tot = lanered(e0 + e1 + e2 + e3, jnp.add)
                rcp = 1.0 / tot
                lp_v[pl.ds(o, 16)] = e0 * rcp
                lp_v[pl.ds(o + 16, 16)] = e1 * rcp
                lp_v[pl.ds(o + 32, 16)] = e2 * rcp
                lp_v[pl.ds(o + 48, 16)] = e3 * rcp
                lc_vec = jnp.where(io == j, local, lc_vec)
            ev = eid_v[pl.ds(g * 16, 16)]
            gp_v[pl.ds(g * 16, 16)] = (lc_vec + ev * CLS).astype(jnp.float32)
            return carry

        lax.fori_loop(0, CHUNK // 16, group, 0)
        pltpu.sync_copy(lp_v, lp_hbm.at[pl.ds(base * CLS, CHUNK * CLS)])
        pltpu.sync_copy(gp_v, gp_hbm.at[pl.ds(base, CHUNK)])

    return k(sel1, eid1)


@jax.jit
def _moe_fused(x4, W_bb, b_bb2, W_c, b_c2, W_e, b_e):
    grid = (B // BT,)
    coarse, eid, sel = pl.pallas_call(
        _moe_block,
        grid=grid,
        in_specs=[
            pl.BlockSpec((BT, 1, 1, D_IN), lambda i: (i, 0, 0, 0)),
            pl.BlockSpec((D_IN, FEAT), lambda i: (0, 0)),
            pl.BlockSpec((1, FEAT), lambda i: (0, 0)),
            pl.BlockSpec((FEAT, E), lambda i: (0, 0)),
            pl.BlockSpec((1, E), lambda i: (0, 0)),
            pl.BlockSpec((E, FEAT, CLS), lambda i: (0, 0, 0)),
            pl.BlockSpec((1, E * CLS), lambda i: (0, 0)),
        ],
        out_specs=[
            pl.BlockSpec((BT, E), lambda i: (i, 0)),
            pl.BlockSpec((BT, 1), lambda i: (i, 0)),
            pl.BlockSpec((BT, CLS), lambda i: (i, 0)),
        ],
        out_shape=[
            jax.ShapeDtypeStruct((B, E), jnp.float32),
            jax.ShapeDtypeStruct((B, 1), jnp.int32),
            jax.ShapeDtypeStruct((B, CLS), jnp.float32),
        ],
        scratch_shapes=[pltpu.VMEM((FEAT, E * CLS), jnp.float32)],
        compiler_params=pltpu.CompilerParams(
            dimension_semantics=("arbitrary",),
        ),
    )(x4, W_bb, b_bb2, W_c, b_c2, W_e, b_e)

    eid1 = eid.reshape(B)
    lp1, gp = _sc_epilogue(sel.reshape(B * CLS), eid1)
    return coarse, eid1, lp1.reshape(B, CLS), gp


def kernel(x, W_bb, b_bb, W_c, b_c, W_e, b_e):
    return _moe_fused(x, W_bb, b_bb.reshape(1, FEAT),
                      W_c, b_c.reshape(1, E), W_e,
                      b_e.reshape(1, E * CLS))


# speedup vs baseline: 1.0379x; 1.0379x over previous
import functools

import jax
import jax.numpy as jnp
from jax import lax
from jax.experimental import pallas as pl
from jax.experimental.pallas import tpu as pltpu
from jax.experimental.pallas import tpu_sc as plsc

B = 8192
D_IN = 1024
FEAT = 512
E = 16
CLS = 64
BT = 1024
SUB = 256

NWORK = 32
CHUNK = B // NWORK


def _moe_block(x_ref, wbb_ref, bbb_ref, wc_ref, bc_ref, we_ref, be_ref,
               coarse_ref, eid_ref, sel_ref, wcat_ref):
    @pl.when(pl.program_id(0) == 0)
    def _init():
        for e in range(E):
            wcat_ref[:, e * CLS:(e + 1) * CLS] = we_ref[e]

    wbb = wbb_ref[...]
    wc = wc_ref[...]
    wcat = wcat_ref[...]
    for s in range(BT // SUB):
        r0 = s * SUB
        xb = x_ref[r0:r0 + SUB].reshape(SUB, D_IN)
        sf = jnp.maximum(jnp.dot(xb, wbb,
                                 preferred_element_type=jnp.float32)
                         + bbb_ref[...], 0.0)
        coarse = jnp.dot(sf, wc,
                         preferred_element_type=jnp.float32) + bc_ref[...]
        coarse_ref[r0:r0 + SUB] = coarse
        eid = jnp.argmax(coarse, axis=1).astype(jnp.int32)
        eid_ref[r0:r0 + SUB] = eid[:, None]

        all_l = jnp.dot(sf, wcat,
                        preferred_element_type=jnp.float32) + be_ref[...]

        pid = eid >> 1
        sel2 = all_l[:, 0:2 * CLS]
        for p in range(1, E // 2):
            sel2 = jnp.where((pid == p)[:, None],
                             all_l[:, p * 2 * CLS:(p + 1) * 2 * CLS], sel2)
        sel_ref[r0:r0 + SUB] = jnp.where((eid & 1 == 1)[:, None],
                                         sel2[:, CLS:2 * CLS],
                                         sel2[:, 0:CLS])


def _sc_epilogue(sel1, eid1):
    mesh = plsc.VectorSubcoreMesh(core_axis_name="c", subcore_axis_name="s")

    @functools.partial(
        pl.kernel, mesh=mesh,
        out_type=[jax.ShapeDtypeStruct((B * CLS,), jnp.float32),
                  jax.ShapeDtypeStruct((B,), jnp.float32)],
        scratch_types=[pltpu.VMEM((CHUNK * CLS,), jnp.float32),
                       pltpu.VMEM((CHUNK,), jnp.int32),
                       pltpu.VMEM((CHUNK * CLS,), jnp.float32),
                       pltpu.VMEM((CHUNK,), jnp.float32)],
    )
    def k(sel_hbm, eid_hbm, lp_hbm, gp_hbm, sel_v, eid_v, lp_v, gp_v):
        wid = lax.axis_index("s") * 2 + lax.axis_index("c")
        base = wid * CHUNK
        pltpu.sync_copy(sel_hbm.at[pl.ds(base * CLS, CHUNK * CLS)], sel_v)
        pltpu.sync_copy(eid_hbm.at[pl.ds(base, CHUNK)], eid_v)
        io = lax.broadcasted_iota(jnp.int32, (16,), 0)
        big = jnp.int32(1 << 20)

        def lanered(v, op):
            for sh in (8, 4, 2, 1):
                perm = io ^ sh
                shuf = lax.gather(
                    v, perm[:, None],
                    lax.GatherDimensionNumbers(offset_dims=(),
                                               collapsed_slice_dims=(0,),
                                               start_index_map=(0,)),
                    slice_sizes=(1,),
                    mode=lax.GatherScatterMode.PROMISE_IN_BOUNDS)
                v = op(v, shuf)
            return v

        def bcast(v, j):
            return lax.gather(
                v, jnp.full((16, 1), j, jnp.int32),
                lax.GatherDimensionNumbers(offset_dims=(),
                                           collapsed_slice_dims=(0,),
                                           start_index_map=(0,)),
                slice_sizes=(1,),
                mode=lax.GatherScatterMode.PROMISE_IN_BOUNDS)

        def group(g, carry):
            lc_vec = jnp.zeros((16,), jnp.int32)
            tot_vec = jnp.zeros((16,), jnp.float32)
            for j in range(16):
                o = (g * 16 + j) * CLS
                v0 = sel_v[pl.ds(o, 16)]
                v1 = sel_v[pl.ds(o + 16, 16)]
                v2 = sel_v[pl.ds(o + 32, 16)]
                v3 = sel_v[pl.ds(o + 48, 16)]
                m = lanered(jnp.maximum(jnp.maximum(v0, v1),
                                        jnp.maximum(v2, v3)), jnp.maximum)
                i0 = jnp.where(v0 == m, io, big)
                i1 = jnp.where(v1 == m, io + 16, big)
                i2 = jnp.where(v2 == m, io + 32, big)
                i3 = jnp.where(v3 == m, io + 48, big)
                local = lanered(jnp.minimum(jnp.minimum(i0, i1),
                                            jnp.minimum(i2, i3)), jnp.minimum)
                e0 = jnp.exp(v0 - m)
                e1 = jnp.exp(v1 - m)
                e2 = jnp.exp(v2 - m)
                e3 = jnp.exp(v3 - m)
                tot = lanered(e0 + e1 + e2 + e3, jnp.add)
                lp_v[pl.ds(o, 16)] = e0
                lp_v[pl.ds(o + 16, 16)] = e1
                lp_v[pl.ds(o + 32, 16)] = e2
                lp_v[pl.ds(o + 48, 16)] = e3
                lc_vec = jnp.where(io == j, local, lc_vec)
                tot_vec = jnp.where(io == j, tot, tot_vec)
            rcp_vec = 1.0 / tot_vec
            for j in range(16):
                o = (g * 16 + j) * CLS
                r = bcast(rcp_vec, j)
                lp_v[pl.ds(o, 16)] = lp_v[pl.ds(o, 16)] * r
                lp_v[pl.ds(o + 16, 16)] = lp_v[pl.ds(o + 16, 16)] * r
                lp_v[pl.ds(o + 32, 16)] = lp_v[pl.ds(o + 32, 16)] * r
                lp_v[pl.ds(o + 48, 16)] = lp_v[pl.ds(o + 48, 16)] * r
            ev = eid_v[pl.ds(g * 16, 16)]
            gp_v[pl.ds(g * 16, 16)] = (lc_vec + ev * CLS).astype(jnp.float32)
            return carry

        lax.fori_loop(0, CHUNK // 16, group, 0)
        pltpu.sync_copy(lp_v, lp_hbm.at[pl.ds(base * CLS, CHUNK * CLS)])
        pltpu.sync_copy(gp_v, gp_hbm.at[pl.ds(base, CHUNK)])

    return k(sel1, eid1)


@jax.jit
def _moe_fused(x4, W_bb, b_bb2, W_c, b_c2, W_e, b_e):
    grid = (B // BT,)
    coarse, eid, sel = pl.pallas_call(
        _moe_block,
        grid=grid,
        in_specs=[
            pl.BlockSpec((BT, 1, 1, D_IN), lambda i: (i, 0, 0, 0)),
            pl.BlockSpec((D_IN, FEAT), lambda i: (0, 0)),
            pl.BlockSpec((1, FEAT), lambda i: (0, 0)),
            pl.BlockSpec((FEAT, E), lambda i: (0, 0)),
            pl.BlockSpec((1, E), lambda i: (0, 0)),
            pl.BlockSpec((E, FEAT, CLS), lambda i: (0, 0, 0)),
            pl.BlockSpec((1, E * CLS), lambda i: (0, 0)),
        ],
        out_specs=[
            pl.BlockSpec((BT, E), lambda i: (i, 0)),
            pl.BlockSpec((BT, 1), lambda i: (i, 0)),
            pl.BlockSpec((BT, CLS), lambda i: (i, 0)),
        ],
        out_shape=[
            jax.ShapeDtypeStruct((B, E), jnp.float32),
            jax.ShapeDtypeStruct((B, 1), jnp.int32),
            jax.ShapeDtypeStruct((B, CLS), jnp.float32),
        ],
        scratch_shapes=[pltpu.VMEM((FEAT, E * CLS), jnp.float32)],
        compiler_params=pltpu.CompilerParams(
            dimension_semantics=("arbitrary",),
        ),
    )(x4, W_bb, b_bb2, W_c, b_c2, W_e, b_e)

    eid1 = eid.reshape(B)
    lp1, gp = _sc_epilogue(sel.reshape(B * CLS), eid1)
    return coarse, eid1, lp1.reshape(B, CLS), gp


def kernel(x, W_bb, b_bb, W_c, b_c, W_e, b_e):
    return _moe_fused(x, W_bb, b_bb.reshape(1, FEAT),
                      W_c, b_c.reshape(1, E), W_e,
                      b_e.reshape(1, E * CLS))
